# baseline (speedup 1.0000x reference)
import jax
import jax.numpy as jnp
from jax import lax
from jax.experimental import pallas as pl
from jax.experimental.pallas import tpu as pltpu
from jax.experimental.pallas import tpu_sc as plsc

B = 16384
D = 64
PANEL = 512
HPAN = PANEL // 2
NPAN = 196
VP = NPAN * HPAN
NC = 2
NS = 16
NW = NC * NS
BPW = B // NW
L = 16
CHUNK = 128
NCH = BPW // CHUNK
NG = CHUNK // L


def _pack_body(t_ref, o_ref):
    y = t_ref[...].T
    o_ref[:, 0:D] = y[0:HPAN, :]
    o_ref[:, D:2 * D] = y[HPAN:PANEL, :]


def _pack_pairs(table):
    tt = table.T
    return pl.pallas_call(
        _pack_body,
        grid=(NPAN,),
        in_specs=[pl.BlockSpec((D, PANEL), lambda j: (0, j))],
        out_specs=pl.BlockSpec((HPAN, 2 * D), lambda j: (j, 0)),
        out_shape=jax.ShapeDtypeStruct((VP, 2 * D), jnp.float32),
    )(tt)


def _sc_body(u_ids_hbm, i_ids_hbm, ent_map_hbm, user_hbm, item_hbm, ent_hbm,
             out_hbm, uid_v, iid_v, eid_v, upid_v, ipid_v, epid_v,
             u_rows, i_rows, e_rows, out_v, sem_u, sem_i, sem_e):
    wid = lax.axis_index("s") * NC + lax.axis_index("c")
    base = wid * BPW

    pltpu.sync_copy(u_ids_hbm.at[pl.ds(base, BPW)], uid_v)
    pltpu.sync_copy(i_ids_hbm.at[pl.ds(base, BPW)], iid_v)
    pltpu.async_copy(ent_map_hbm.at[iid_v], eid_v, sem_e).wait()

    def pid_body(g, _):
        sl = pl.ds(g * L, L)
        u = uid_v[sl]
        i = iid_v[sl]
        e = eid_v[sl]
        upid_v[sl] = lax.shift_right_logical(u, 9) * HPAN + (u & (HPAN - 1))
        ipid_v[sl] = lax.shift_right_logical(i, 9) * HPAN + (i & (HPAN - 1))
        epid_v[sl] = lax.shift_right_logical(e, 9) * HPAN + (e & (HPAN - 1))
        return 0

    lax.fori_loop(0, BPW // L, pid_body, 0)

    lane = lax.iota(jnp.int32, L)

    def chunk_body(ch, _):
        c0 = ch * CHUNK
        cp_u = pltpu.async_copy(
            user_hbm.at[upid_v.at[pl.ds(c0, CHUNK)]], u_rows, sem_u)
        cp_i = pltpu.async_copy(
            item_hbm.at[ipid_v.at[pl.ds(c0, CHUNK)]], i_rows, sem_i)
        cp_e = pltpu.async_copy(
            ent_hbm.at[epid_v.at[pl.ds(c0, CHUNK)]], e_rows, sem_e)
        cp_u.wait()
        cp_i.wait()
        cp_e.wait()

        def group_body(g, _):
            b0 = g * L
            gsl = pl.ds(c0 + b0, L)
            uoff = (lax.shift_right_logical(uid_v[gsl], 8) & 1) * D
            ioff = (lax.shift_right_logical(iid_v[gsl], 8) & 1) * D
            eoff = (lax.shift_right_logical(eid_v[gsl], 8) & 1) * D
            acc = jnp.zeros((L,), jnp.float32)
            for k in range(L):
                b = b0 + k
                uo = uoff[k]
                io = ioff[k]
                eo = eoff[k]
                p = jnp.zeros((L,), jnp.float32)
                for j in range(D // L):
                    u = u_rows[b, pl.ds(uo + j * L, L)]
                    iv = i_rows[b, pl.ds(io + j * L, L)]
                    ev = e_rows[b, pl.ds(eo + j * L, L)]
                    p = p + u * (iv + ev)
                acc = jnp.where(lane == k, jnp.sum(p), acc)
            out_v[pl.ds(c0 + b0, L)] = acc
            return 0

        lax.fori_loop(0, NG, group_body, 0)
        return 0

    lax.fori_loop(0, NCH, chunk_body, 0)
    pltpu.sync_copy(out_v, out_hbm.at[pl.ds(base, BPW)])


def kernel(u_ids, i_ids, ent_map, user_emb, item_emb, ent_emb):
    up = _pack_pairs(user_emb)
    ip = _pack_pairs(item_emb)
    ep = _pack_pairs(ent_emb)

    mesh = plsc.VectorSubcoreMesh(core_axis_name="c", subcore_axis_name="s")
    f = pl.kernel(
        _sc_body,
        out_type=jax.ShapeDtypeStruct((B,), jnp.float32),
        mesh=mesh,
        compiler_params=pltpu.CompilerParams(
            needs_layout_passes=False, use_tc_tiling_on_sc=True),
        scratch_types=[
            pltpu.VMEM((BPW,), jnp.int32),
            pltpu.VMEM((BPW,), jnp.int32),
            pltpu.VMEM((BPW,), jnp.int32),
            pltpu.VMEM((BPW,), jnp.int32),
            pltpu.VMEM((BPW,), jnp.int32),
            pltpu.VMEM((BPW,), jnp.int32),
            pltpu.VMEM((CHUNK, 2 * D), jnp.float32),
            pltpu.VMEM((CHUNK, 2 * D), jnp.float32),
            pltpu.VMEM((CHUNK, 2 * D), jnp.float32),
            pltpu.VMEM((BPW,), jnp.float32),
            pltpu.SemaphoreType.DMA,
            pltpu.SemaphoreType.DMA,
            pltpu.SemaphoreType.DMA,
        ],
    )
    return f(u_ids.astype(jnp.int32), i_ids.astype(jnp.int32),
             ent_map.astype(jnp.int32), up, ip, ep)


# speedup vs baseline: 3.4380x; 3.4380x over previous
import jax
import jax.numpy as jnp
from jax import lax
from jax.experimental import pallas as pl
from jax.experimental.pallas import tpu as pltpu
from jax.experimental.pallas import tpu_sc as plsc

B = 16384
D = 64
PANEL = 16384
HPAN = PANEL // 2
NPAN = 7
VP = NPAN * HPAN
SH = PANEL.bit_length() - 1
NC = 2
NS = 16
NW = NC * NS
BPW = B // NW
L = 16
CHUNK = 128
NCH = BPW // CHUNK
NG = CHUNK // L


def _pack_body(t_ref, o_ref):
    y = t_ref[...].T
    o_ref[:, 0:D] = y[0:HPAN, :]
    o_ref[:, D:2 * D] = y[HPAN:PANEL, :]


def _pack_pairs(table):
    tt = table.T
    return pl.pallas_call(
        _pack_body,
        grid=(NPAN,),
        in_specs=[pl.BlockSpec((D, PANEL), lambda j: (0, j))],
        out_specs=pl.BlockSpec((HPAN, 2 * D), lambda j: (j, 0)),
        out_shape=jax.ShapeDtypeStruct((VP, 2 * D), jnp.float32),
    )(tt)


def _pids(ids):
    return lax.shift_right_logical(ids, SH) * HPAN + (ids & (HPAN - 1))


def _halves(ids):
    return (lax.shift_right_logical(ids, SH - 1) & 1) * D


def _sc_a_body(u_ids_hbm, i_ids_hbm, ent_map_hbm, user_hbm, item_hbm,
               part_hbm, eids_hbm, uflat_hbm,
               uid_v, iid_v, eid_v, upid_v, ipid_v,
               u_rows, u_rows2, i_rows, i_rows2, ustage, out_v,
               sem_u, sem_u2, sem_i, sem_i2, sem_m):
    wid = lax.axis_index("s") * NC + lax.axis_index("c")
    base = wid * BPW

    pltpu.sync_copy(u_ids_hbm.at[pl.ds(base, BPW)], uid_v)
    pltpu.sync_copy(i_ids_hbm.at[pl.ds(base, BPW)], iid_v)
    pltpu.async_copy(ent_map_hbm.at[iid_v], eid_v, sem_m).wait()
    pltpu.sync_copy(eid_v, eids_hbm.at[pl.ds(base, BPW)])

    def pid_body(g, _):
        sl = pl.ds(g * L, L)
        upid_v[sl] = _pids(uid_v[sl])
        ipid_v[sl] = _pids(iid_v[sl])
        return 0

    lax.fori_loop(0, BPW // L, pid_body, 0)

    lane = lax.iota(jnp.int32, L)
    u_bufs = (u_rows, u_rows2)
    i_bufs = (i_rows, i_rows2)
    sems_u = (sem_u, sem_u2)
    sems_i = (sem_i, sem_i2)

    def issue(ch):
        p = ch % 2
        c0 = ch * CHUNK
        return (
            pltpu.async_copy(
                user_hbm.at[upid_v.at[pl.ds(c0, CHUNK)]], u_bufs[p], sems_u[p]),
            pltpu.async_copy(
                item_hbm.at[ipid_v.at[pl.ds(c0, CHUNK)]], i_bufs[p], sems_i[p]),
        )

    def compute(ch):
        p = ch % 2
        c0 = ch * CHUNK
        u_rows_c = u_bufs[p]
        i_rows_c = i_bufs[p]

        def group_body(g, _):
            b0 = g * L
            gsl = pl.ds(c0 + b0, L)
            uoff = _halves(uid_v[gsl])
            ioff = _halves(iid_v[gsl])
            acc = jnp.zeros((L,), jnp.float32)
            for k in range(L):
                b = b0 + k
                uo = uoff[k]
                io = ioff[k]
                pv = jnp.zeros((L,), jnp.float32)
                for j in range(D // L):
                    u = u_rows_c[b, pl.ds(uo + j * L, L)]
                    iv = i_rows_c[b, pl.ds(io + j * L, L)]
                    pv = pv + u * iv
                    ustage[pl.ds(b * D + j * L, L)] = u
                acc = jnp.where(lane == k, jnp.sum(pv), acc)
            out_v[pl.ds(c0 + b0, L)] = acc
            return 0

        lax.fori_loop(0, NG, group_body, 0)
        pltpu.sync_copy(
            ustage, uflat_hbm.at[pl.ds((base + c0) * D, CHUNK * D)])

    cps = issue(0)
    for ch in range(NCH):
        nxt = issue(ch + 1) if ch + 1 < NCH else None
        for cp in cps:
            cp.wait()
        compute(ch)
        cps = nxt
    pltpu.sync_copy(out_v, part_hbm.at[pl.ds(base, BPW)])


def _sc_b_body(eids_hbm, part_hbm, uflat_hbm, ent_hbm, out_hbm,
               eid_v, epid_v, e_rows, e_rows2, ubuf, part_v, out_v,
               sem_e, sem_e2, sem_p):
    wid = lax.axis_index("s") * NC + lax.axis_index("c")
    base = wid * BPW

    pltpu.sync_copy(eids_hbm.at[pl.ds(base, BPW)], eid_v)
    cp_p = pltpu.async_copy(part_hbm.at[pl.ds(base, BPW)], part_v, sem_p)

    def pid_body(g, _):
        sl = pl.ds(g * L, L)
        epid_v[sl] = _pids(eid_v[sl])
        return 0

    lax.fori_loop(0, BPW // L, pid_body, 0)

    lane = lax.iota(jnp.int32, L)
    e_bufs = (e_rows, e_rows2)
    sems_e = (sem_e, sem_e2)

    def issue(ch):
        p = ch % 2
        c0 = ch * CHUNK
        return (
            pltpu.async_copy(
                ent_hbm.at[epid_v.at[pl.ds(c0, CHUNK)]], e_bufs[p], sems_e[p]),
            pltpu.async_copy(
                uflat_hbm.at[pl.ds((base + c0) * D, CHUNK * D)],
                ubuf.at[pl.ds(p * CHUNK * D, CHUNK * D)], sems_e[p]),
        )

    def compute(ch):
        p = ch % 2
        c0 = ch * CHUNK
        e_rows_c = e_bufs[p]
        ub = p * CHUNK * D

        def group_body(g, _):
            b0 = g * L
            gsl = pl.ds(c0 + b0, L)
            eoff = _halves(eid_v[gsl])
            acc = jnp.zeros((L,), jnp.float32)
            for k in range(L):
                b = b0 + k
                eo = eoff[k]
                pv = jnp.zeros((L,), jnp.float32)
                for j in range(D // L):
                    u = ubuf[pl.ds(ub + b * D + j * L, L)]
                    ev = e_rows_c[b, pl.ds(eo + j * L, L)]
                    pv = pv + u * ev
                acc = jnp.where(lane == k, jnp.sum(pv), acc)
            out_v[pl.ds(c0 + b0, L)] = acc + part_v[pl.ds(c0 + b0, L)]
            return 0

        lax.fori_loop(0, NG, group_body, 0)

    cps = issue(0)
    cp_p.wait()
    for ch in range(NCH):
        nxt = issue(ch + 1) if ch + 1 < NCH else None
        for cp in cps:
            cp.wait()
        compute(ch)
        cps = nxt
    pltpu.sync_copy(out_v, out_hbm.at[pl.ds(base, BPW)])


def kernel(u_ids, i_ids, ent_map, user_emb, item_emb, ent_emb):
    up = _pack_pairs(user_emb)
    ip = _pack_pairs(item_emb)
    ep = _pack_pairs(ent_emb)

    mesh = plsc.VectorSubcoreMesh(core_axis_name="c", subcore_axis_name="s")
    cparams = pltpu.CompilerParams(
        needs_layout_passes=False, use_tc_tiling_on_sc=True)

    fa = pl.kernel(
        _sc_a_body,
        out_type=(
            jax.ShapeDtypeStruct((B,), jnp.float32),
            jax.ShapeDtypeStruct((B,), jnp.int32),
            jax.ShapeDtypeStruct((B * D,), jnp.float32),
        ),
        mesh=mesh,
        compiler_params=cparams,
        scratch_types=[
            pltpu.VMEM((BPW,), jnp.int32),
            pltpu.VMEM((BPW,), jnp.int32),
            pltpu.VMEM((BPW,), jnp.int32),
            pltpu.VMEM((BPW,), jnp.int32),
            pltpu.VMEM((BPW,), jnp.int32),
            pltpu.VMEM((CHUNK, 2 * D), jnp.float32),
            pltpu.VMEM((CHUNK, 2 * D), jnp.float32),
            pltpu.VMEM((CHUNK, 2 * D), jnp.float32),
            pltpu.VMEM((CHUNK, 2 * D), jnp.float32),
            pltpu.VMEM((CHUNK * D,), jnp.float32),
            pltpu.VMEM((BPW,), jnp.float32),
            pltpu.SemaphoreType.DMA,
            pltpu.SemaphoreType.DMA,
            pltpu.SemaphoreType.DMA,
            pltpu.SemaphoreType.DMA,
            pltpu.SemaphoreType.DMA,
        ],
    )
    part, eids, uflat = fa(u_ids.astype(jnp.int32), i_ids.astype(jnp.int32),
                           ent_map.astype(jnp.int32), up, ip)

    fb = pl.kernel(
        _sc_b_body,
        out_type=jax.ShapeDtypeStruct((B,), jnp.float32),
        mesh=mesh,
        compiler_params=cparams,
        scratch_types=[
            pltpu.VMEM((BPW,), jnp.int32),
            pltpu.VMEM((BPW,), jnp.int32),
            pltpu.VMEM((CHUNK, 2 * D), jnp.float32),
            pltpu.VMEM((CHUNK, 2 * D), jnp.float32),
            pltpu.VMEM((2 * CHUNK * D,), jnp.float32),
            pltpu.VMEM((BPW,), jnp.float32),
            pltpu.VMEM((BPW,), jnp.float32),
            pltpu.SemaphoreType.DMA,
            pltpu.SemaphoreType.DMA,
            pltpu.SemaphoreType.DMA,
        ],
    )
    return fb(eids, part, uflat, ep)
